# baseline (speedup 1.0000x reference)
import jax
import jax.numpy as jnp
from jax import lax
from jax.experimental import pallas as pl
from jax.experimental.pallas import tpu as pltpu
from jax.experimental.pallas import tpu_sc as plsc

PAD_ID = 0
LANES = 16
NUM_CORES = 2
NUM_SUBCORES = 16
BN = 16384


def _project(emb_table, w2):
    V, D = emb_table.shape
    tt = emb_table.T
    nb = pl.cdiv(V, BN)

    def body(w_ref, t_ref, o_ref):
        o_ref[...] = jnp.dot(w_ref[...], t_ref[...],
                             preferred_element_type=jnp.float32)

    return pl.pallas_call(
        body,
        grid=(nb,),
        in_specs=[pl.BlockSpec((2, D), lambda i: (0, 0)),
                  pl.BlockSpec((D, BN), lambda i: (0, i))],
        out_specs=pl.BlockSpec((2, BN), lambda i: (0, i)),
        out_shape=jax.ShapeDtypeStruct((2, V), jnp.float32),
    )(w2, tt)


def _sc_scorer(B, L, V, LP1):
    RPT = B // NUM_SUBCORES
    NCH = (LP1 + LANES - 1) // LANES
    NTOK = RPT * LP1
    INV_PAD = ((L + 1 + 63) // 64) * 64

    mesh = plsc.VectorSubcoreMesh(
        core_axis_name="c", subcore_axis_name="s")

    def body(tok_hbm, p_hbm, inv_hbm, out_hbm,
             idx_v, vals, inv_v, stage, p_sh, sem):
        cid = lax.axis_index("c")
        sid = lax.axis_index("s")
        tok_base = cid * B * LP1 + sid * NTOK
        pltpu.sync_copy(inv_hbm, inv_v)
        pltpu.sync_copy(tok_hbm.at[pl.ds(tok_base, NTOK)], idx_v)

        @pl.when(sid == 0)
        def _():
            pltpu.sync_copy(p_hbm.at[cid], p_sh)

        plsc.subcore_barrier()

        pltpu.async_copy(p_sh.at[idx_v], vals, sem).wait()

        lane = jnp.arange(LANES, dtype=jnp.int32)
        zeros_f = jnp.zeros((LANES,), jnp.float32)
        zeros_i = jnp.zeros((LANES,), jnp.int32)
        struct = [jnp.arange(k * LANES, (k + 1) * LANES) < LP1
                  for k in range(NCH)]

        def do_row(b, carry):
            acc = zeros_f
            cnt = zeros_i
            for k in range(NCH):
                pos = jnp.minimum(lane + (b * LP1 + k * LANES), NTOK - 1)
                tok = plsc.load_gather(idx_v, [pos])
                val = plsc.load_gather(vals, [pos])
                m = jnp.asarray(struct[k]) & (tok != PAD_ID)
                acc = acc + jnp.where(m, val, 0.0)
                cnt = cnt + jnp.where(m, 1, 0).astype(jnp.int32)

            inv = plsc.load_gather(inv_v, [zeros_i + jnp.sum(cnt)])
            score = (zeros_f + jnp.sum(acc)) * inv
            plsc.store_scatter(stage, [b * LANES + lane], score)
            return carry

        lax.fori_loop(0, RPT, do_row, jnp.int32(0))

        pltpu.sync_copy(
            stage,
            out_hbm.at[pl.ds(cid * B * LANES + sid * RPT * LANES,
                             RPT * LANES)])

    return pl.kernel(
        body,
        out_type=jax.ShapeDtypeStruct((2 * B * LANES,), jnp.float32),
        mesh=mesh,
        compiler_params=pltpu.CompilerParams(
            needs_layout_passes=False, use_tc_tiling_on_sc=False),
        scratch_types=[
            pltpu.VMEM((NTOK,), jnp.int32),
            pltpu.VMEM((NTOK,), jnp.float32),
            pltpu.VMEM((INV_PAD,), jnp.float32),
            pltpu.VMEM((RPT * LANES,), jnp.float32),
            pltpu.VMEM_SHARED((V,), jnp.float32),
            pltpu.SemaphoreType.DMA,
        ],
    )


def kernel(candidates, head_mentions, emb_table, fc_w, fc_b):
    B, L = candidates.shape
    V, D = emb_table.shape

    w2 = jnp.stack((fc_w[:D, 0], fc_w[D:, 0]))
    p = _project(emb_table, w2)

    INV_PAD = ((L + 1 + 63) // 64) * 64
    inv_tab = jnp.where(
        jnp.arange(INV_PAD) <= L,
        1.0 / (jnp.float32(L) * jnp.arange(INV_PAD, dtype=jnp.float32)),
        0.0).astype(jnp.float32)

    LP1 = ((L + 7) // 8) * 8
    toks = jnp.stack((candidates, head_mentions))
    toks = jnp.pad(toks, ((0, 0), (0, 0), (0, LP1 - L))).reshape(-1)

    halves = _sc_scorer(B, L, V, LP1)(toks, p, inv_tab)
    o = halves.reshape(2, B, LANES)
    return (o[0, :, :1] + o[1, :, :1]) + fc_b


# speedup vs baseline: 1.0061x; 1.0061x over previous
import jax
import jax.numpy as jnp
from jax import lax
from jax.experimental import pallas as pl
from jax.experimental.pallas import tpu as pltpu
from jax.experimental.pallas import tpu_sc as plsc

PAD_ID = 0
LANES = 16
NUM_CORES = 2
NUM_SUBCORES = 16
BN = 8192


def _project(emb_table, w2):
    V, D = emb_table.shape
    tt = emb_table.T
    nb = pl.cdiv(V, BN)

    def body(w_ref, t_ref, o_ref):
        o_ref[...] = jnp.dot(w_ref[...], t_ref[...],
                             preferred_element_type=jnp.float32)

    return pl.pallas_call(
        body,
        grid=(nb,),
        in_specs=[pl.BlockSpec((2, D), lambda i: (0, 0)),
                  pl.BlockSpec((D, BN), lambda i: (0, i))],
        out_specs=pl.BlockSpec((2, BN), lambda i: (0, i)),
        out_shape=jax.ShapeDtypeStruct((2, V), jnp.float32),
    )(w2, tt)


def _sc_scorer(B, L, V, LP1):
    RPT = B // NUM_SUBCORES
    NCH = (LP1 + LANES - 1) // LANES
    NTOK = RPT * LP1
    INV_PAD = ((L + 1 + 63) // 64) * 64

    mesh = plsc.VectorSubcoreMesh(
        core_axis_name="c", subcore_axis_name="s")

    def body(tok_hbm, p_hbm, inv_hbm, out_hbm,
             idx_v, vals, inv_v, stage, p_sh, sem):
        cid = lax.axis_index("c")
        sid = lax.axis_index("s")
        tok_base = cid * B * LP1 + sid * NTOK
        pltpu.sync_copy(inv_hbm, inv_v)
        pltpu.sync_copy(tok_hbm.at[pl.ds(tok_base, NTOK)], idx_v)

        @pl.when(sid == 0)
        def _():
            pltpu.sync_copy(p_hbm.at[cid], p_sh)

        plsc.subcore_barrier()

        pltpu.async_copy(p_sh.at[idx_v], vals, sem).wait()

        lane = jnp.arange(LANES, dtype=jnp.int32)
        zeros_f = jnp.zeros((LANES,), jnp.float32)
        zeros_i = jnp.zeros((LANES,), jnp.int32)
        struct = [jnp.arange(k * LANES, (k + 1) * LANES) < LP1
                  for k in range(NCH)]

        def do_row(b, carry):
            acc = zeros_f
            cnt = zeros_i
            for k in range(NCH):
                pos = jnp.minimum(lane + (b * LP1 + k * LANES), NTOK - 1)
                tok = plsc.load_gather(idx_v, [pos])
                val = plsc.load_gather(vals, [pos])
                m = jnp.asarray(struct[k]) & (tok != PAD_ID)
                acc = acc + jnp.where(m, val, 0.0)
                cnt = cnt + jnp.where(m, 1, 0).astype(jnp.int32)

            inv = plsc.load_gather(inv_v, [zeros_i + jnp.sum(cnt)])
            score = (zeros_f + jnp.sum(acc)) * inv
            plsc.store_scatter(stage, [b * LANES + lane], score)
            return carry

        lax.fori_loop(0, RPT, do_row, jnp.int32(0))

        pltpu.sync_copy(
            stage,
            out_hbm.at[pl.ds(cid * B * LANES + sid * RPT * LANES,
                             RPT * LANES)])

    return pl.kernel(
        body,
        out_type=jax.ShapeDtypeStruct((2 * B * LANES,), jnp.float32),
        mesh=mesh,
        compiler_params=pltpu.CompilerParams(
            needs_layout_passes=False, use_tc_tiling_on_sc=False),
        scratch_types=[
            pltpu.VMEM((NTOK,), jnp.int32),
            pltpu.VMEM((NTOK,), jnp.float32),
            pltpu.VMEM((INV_PAD,), jnp.float32),
            pltpu.VMEM((RPT * LANES,), jnp.float32),
            pltpu.VMEM_SHARED((V,), jnp.float32),
            pltpu.SemaphoreType.DMA,
        ],
    )


def kernel(candidates, head_mentions, emb_table, fc_w, fc_b):
    B, L = candidates.shape
    V, D = emb_table.shape

    w2 = jnp.stack((fc_w[:D, 0], fc_w[D:, 0]))
    p = _project(emb_table, w2)

    INV_PAD = ((L + 1 + 63) // 64) * 64
    inv_tab = jnp.where(
        jnp.arange(INV_PAD) <= L,
        1.0 / (jnp.float32(L) * jnp.arange(INV_PAD, dtype=jnp.float32)),
        0.0).astype(jnp.float32)

    LP1 = ((L + 7) // 8) * 8
    toks = jnp.stack((candidates, head_mentions))
    toks = jnp.pad(toks, ((0, 0), (0, 0), (0, LP1 - L))).reshape(-1)

    halves = _sc_scorer(B, L, V, LP1)(toks, p, inv_tab)
    o = halves.reshape(2, B, LANES)
    return (o[0, :, :1] + o[1, :, :1]) + fc_b


# speedup vs baseline: 1.0078x; 1.0017x over previous
import jax
import jax.numpy as jnp
from jax import lax
from jax.experimental import pallas as pl
from jax.experimental.pallas import tpu as pltpu
from jax.experimental.pallas import tpu_sc as plsc

PAD_ID = 0
LANES = 16
NUM_CORES = 2
NUM_SUBCORES = 16
BN = 6144


def _project(emb_table, w2):
    V, D = emb_table.shape
    tt = emb_table.T
    nb = pl.cdiv(V, BN)

    def body(w_ref, t_ref, o_ref):
        o_ref[...] = jnp.dot(w_ref[...], t_ref[...],
                             preferred_element_type=jnp.float32)

    return pl.pallas_call(
        body,
        grid=(nb,),
        in_specs=[pl.BlockSpec((2, D), lambda i: (0, 0)),
                  pl.BlockSpec((D, BN), lambda i: (0, i))],
        out_specs=pl.BlockSpec((2, BN), lambda i: (0, i)),
        out_shape=jax.ShapeDtypeStruct((2, V), jnp.float32),
    )(w2, tt)


def _sc_scorer(B, L, V, LP1):
    RPT = B // NUM_SUBCORES
    NCH = (LP1 + LANES - 1) // LANES
    NTOK = RPT * LP1
    INV_PAD = ((L + 1 + 63) // 64) * 64

    mesh = plsc.VectorSubcoreMesh(
        core_axis_name="c", subcore_axis_name="s")

    def body(tok_hbm, p_hbm, inv_hbm, out_hbm,
             idx_v, vals, inv_v, stage, p_sh, sem):
        cid = lax.axis_index("c")
        sid = lax.axis_index("s")
        tok_base = cid * B * LP1 + sid * NTOK
        pltpu.sync_copy(inv_hbm, inv_v)
        pltpu.sync_copy(tok_hbm.at[pl.ds(tok_base, NTOK)], idx_v)

        @pl.when(sid == 0)
        def _():
            pltpu.sync_copy(p_hbm.at[cid], p_sh)

        plsc.subcore_barrier()

        pltpu.async_copy(p_sh.at[idx_v], vals, sem).wait()

        lane = jnp.arange(LANES, dtype=jnp.int32)
        zeros_f = jnp.zeros((LANES,), jnp.float32)
        zeros_i = jnp.zeros((LANES,), jnp.int32)
        struct = [jnp.arange(k * LANES, (k + 1) * LANES) < LP1
                  for k in range(NCH)]

        def do_row(b, carry):
            acc = zeros_f
            cnt = zeros_i
            for k in range(NCH):
                pos = jnp.minimum(lane + (b * LP1 + k * LANES), NTOK - 1)
                tok = plsc.load_gather(idx_v, [pos])
                val = plsc.load_gather(vals, [pos])
                m = jnp.asarray(struct[k]) & (tok != PAD_ID)
                acc = acc + jnp.where(m, val, 0.0)
                cnt = cnt + jnp.where(m, 1, 0).astype(jnp.int32)

            inv = plsc.load_gather(inv_v, [zeros_i + jnp.sum(cnt)])
            score = (zeros_f + jnp.sum(acc)) * inv
            plsc.store_scatter(stage, [b * LANES + lane], score)
            return carry

        lax.fori_loop(0, RPT, do_row, jnp.int32(0))

        pltpu.sync_copy(
            stage,
            out_hbm.at[pl.ds(cid * B * LANES + sid * RPT * LANES,
                             RPT * LANES)])

    return pl.kernel(
        body,
        out_type=jax.ShapeDtypeStruct((2 * B * LANES,), jnp.float32),
        mesh=mesh,
        compiler_params=pltpu.CompilerParams(
            needs_layout_passes=False, use_tc_tiling_on_sc=False),
        scratch_types=[
            pltpu.VMEM((NTOK,), jnp.int32),
            pltpu.VMEM((NTOK,), jnp.float32),
            pltpu.VMEM((INV_PAD,), jnp.float32),
            pltpu.VMEM((RPT * LANES,), jnp.float32),
            pltpu.VMEM_SHARED((V,), jnp.float32),
            pltpu.SemaphoreType.DMA,
        ],
    )


def kernel(candidates, head_mentions, emb_table, fc_w, fc_b):
    B, L = candidates.shape
    V, D = emb_table.shape

    w2 = jnp.stack((fc_w[:D, 0], fc_w[D:, 0]))
    p = _project(emb_table, w2)

    INV_PAD = ((L + 1 + 63) // 64) * 64
    inv_tab = jnp.where(
        jnp.arange(INV_PAD) <= L,
        1.0 / (jnp.float32(L) * jnp.arange(INV_PAD, dtype=jnp.float32)),
        0.0).astype(jnp.float32)

    LP1 = ((L + 7) // 8) * 8
    toks = jnp.stack((candidates, head_mentions))
    toks = jnp.pad(toks, ((0, 0), (0, 0), (0, LP1 - L))).reshape(-1)

    halves = _sc_scorer(B, L, V, LP1)(toks, p, inv_tab)
    o = halves.reshape(2, B, LANES)
    return (o[0, :, :1] + o[1, :, :1]) + fc_b
